# baseline (speedup 1.0000x reference)
import functools

import jax
import jax.numpy as jnp
from jax import lax
from jax.experimental import pallas as pl
from jax.experimental.pallas import tpu as pltpu
from jax.experimental.pallas import tpu_sc as plsc

B, LD, LQ, V, D, NF = 4096, 200, 20, 1000000, 64, 4
W = NF + D
N1 = B * LD
N2 = B * LQ
NC, NS = 2, 16
NW = NC * NS
R1 = N1 // NW
R2 = N2 // NW
CH = 512
KI = CH // 128
C1 = R1 // CH
C2 = R2 // CH
G = CH // 4

_mesh = plsc.VectorSubcoreMesh(core_axis_name="c", subcore_axis_name="s")


@functools.partial(
    pl.kernel,
    mesh=_mesh,
    compiler_params=pltpu.CompilerParams(use_tc_tiling_on_sc=False, needs_layout_passes=False),
    out_type=[
        jax.ShapeDtypeStruct((N1 * W,), jnp.float32),
        jax.ShapeDtypeStruct((N2, D), jnp.float32),
    ],
    scratch_types=[
        pltpu.VMEM((CH,), jnp.int32),
        pltpu.VMEM((CH, D), jnp.float32),
        pltpu.VMEM((CH * NF,), jnp.float32),
        pltpu.VMEM((CH * W,), jnp.float32),
        pltpu.SemaphoreType.DMA,
    ],
)
def _emb_gather(x1_hbm, x1f_hbm, x2_hbm, tbl_hbm, out1_hbm, out2_hbm,
                idx_v, emb_v, f_v, out_v, sem):
    wid = lax.axis_index("s") * NC + lax.axis_index("c")
    ii = lax.iota(jnp.int32, 16)
    f_pat = (ii // 4) * W + (ii % 4)
    e_pats = [[ii + (k * W + NF + j * 16) for j in range(4)] for k in range(4)]

    def x1_chunk(c, carry):
        base = wid * R1 + c * CH
        pltpu.sync_copy(x1_hbm.at[pl.ds(base, CH)], idx_v)
        copies = [
            pltpu.async_copy(
                tbl_hbm.at[idx_v.at[pl.ds(j * 128, 128)]],
                emb_v.at[pl.ds(j * 128, 128)],
                sem,
            )
            for j in range(KI)
        ]
        pltpu.sync_copy(x1f_hbm.at[pl.ds(base * NF, CH * NF)], f_v)
        for cp in copies:
            cp.wait()

        def group(g, carry):
            gw = g * 4 * W
            fd = f_v[pl.ds(pl.multiple_of(g * 16, 16), 16)]
            plsc.store_scatter(out_v, [f_pat + gw], fd)
            for k in range(4):
                for j in range(4):
                    ed = emb_v[g * 4 + k, pl.ds(j * 16, 16)]
                    plsc.store_scatter(out_v, [e_pats[k][j] + gw], ed)
            return carry

        lax.fori_loop(0, G, group, 0, unroll=2)
        pltpu.sync_copy(out_v, out1_hbm.at[pl.ds(base * W, CH * W)])
        return carry

    def x2_chunk(c, carry):
        base = wid * R2 + c * CH
        pltpu.sync_copy(x2_hbm.at[pl.ds(base, CH)], idx_v)
        copies = [
            pltpu.async_copy(
                tbl_hbm.at[idx_v.at[pl.ds(j * 128, 128)]],
                emb_v.at[pl.ds(j * 128, 128)],
                sem,
            )
            for j in range(KI)
        ]
        for cp in copies:
            cp.wait()
        pltpu.sync_copy(emb_v, out2_hbm.at[pl.ds(base, CH)])
        return carry

    lax.fori_loop(0, C1, x1_chunk, 0)
    lax.fori_loop(0, C2, x2_chunk, 0)


def kernel(x1, x1_f, x1_pos, x1_ner, x1_mask, x2, x2_mask, sent_lens, emb_table):
    del x1_pos, x1_ner, x1_mask, x2_mask, sent_lens
    x1r = x1.reshape(N1)
    x2r = x2.reshape(N2)
    x1fr = x1_f.reshape(N1 * NF)
    out1, out2 = _emb_gather(x1r, x1fr, x2r, emb_table)
    return out1.reshape(B, LD, W), out2.reshape(B, LQ, D)


# speedup vs baseline: 1.4157x; 1.4157x over previous
import functools

import jax
import jax.numpy as jnp
from jax import lax
from jax.experimental import pallas as pl
from jax.experimental.pallas import tpu as pltpu
from jax.experimental.pallas import tpu_sc as plsc

B, LD, LQ, V, D, NF = 4096, 200, 20, 1000000, 64, 4
W = NF + D
NC, NS = 2, 16
NW = NC * NS
CH = 512
KI = CH // 128
NB = B // CH
Q1 = LD * NB // NW
Q2 = LQ * NB // NW

_mesh = plsc.VectorSubcoreMesh(core_axis_name="c", subcore_axis_name="s")


@functools.partial(
    pl.kernel,
    mesh=_mesh,
    compiler_params=pltpu.CompilerParams(use_tc_tiling_on_sc=False,
                                         needs_layout_passes=False),
    out_type=[
        jax.ShapeDtypeStruct((W, LD, B), jnp.float32),
        jax.ShapeDtypeStruct((LQ, D, B), jnp.float32),
    ],
    scratch_types=[
        pltpu.VMEM((CH,), jnp.int32),
        pltpu.VMEM((CH, D), jnp.float32),
        pltpu.VMEM((D, CH), jnp.float32),
        pltpu.VMEM((NF, CH), jnp.float32),
        pltpu.SemaphoreType.DMA,
    ],
)
def _emb_gather(x1_hbm, x1f_hbm, x2_hbm, tbl_hbm, out1_hbm, out2_hbm,
                idx_v, emb_v, embt_v, f_v, sem):
    wid = lax.axis_index("s") * NC + lax.axis_index("c")
    ii = lax.iota(jnp.int32, 16)

    def gather_chunk(idx_hbm, flat_base):
        pltpu.sync_copy(idx_hbm.at[pl.ds(flat_base, CH)], idx_v)
        copies = [
            pltpu.async_copy(
                tbl_hbm.at[idx_v.at[pl.ds(j * 128, 128)]],
                emb_v.at[pl.ds(j * 128, 128)],
                sem,
            )
            for j in range(KI)
        ]
        for cp in copies:
            cp.wait()

    def transpose_chunk():
        def row(r, carry):
            rr = jnp.full((16,), r, jnp.int32)
            for j in range(4):
                ed = emb_v[r, pl.ds(j * 16, 16)]
                plsc.store_scatter(embt_v, [ii + j * 16, rr], ed)
            return carry

        lax.fori_loop(0, CH, row, 0, unroll=4)

    def x1_chunk(q, carry):
        l = q // NB
        b0 = (q % NB) * CH
        gather_chunk(x1_hbm, l * B + b0)
        transpose_chunk()
        pltpu.sync_copy(x1f_hbm.at[l, :, pl.ds(b0, CH)], f_v)
        pltpu.sync_copy(f_v, out1_hbm.at[pl.ds(0, NF), l, pl.ds(b0, CH)])
        pltpu.sync_copy(embt_v, out1_hbm.at[pl.ds(NF, D), l, pl.ds(b0, CH)])
        return carry

    def x2_chunk(q, carry):
        l = q // NB
        b0 = (q % NB) * CH
        gather_chunk(x2_hbm, l * B + b0)
        transpose_chunk()
        pltpu.sync_copy(embt_v, out2_hbm.at[l, :, pl.ds(b0, CH)])
        return carry

    lax.fori_loop(wid * Q1, (wid + 1) * Q1, x1_chunk, 0)
    lax.fori_loop(wid * Q2, (wid + 1) * Q2, x2_chunk, 0)


def kernel(x1, x1_f, x1_pos, x1_ner, x1_mask, x2, x2_mask, sent_lens, emb_table):
    del x1_pos, x1_ner, x1_mask, x2_mask, sent_lens
    x1t = x1.T.reshape(LD * B)
    x2t = x2.T.reshape(LQ * B)
    x1ft = x1_f.transpose(1, 2, 0)
    out1p, out2p = _emb_gather(x1t, x1ft, x2t, emb_table)
    return out1p.transpose(2, 1, 0), out2p.transpose(2, 0, 1)


# speedup vs baseline: 2.1478x; 1.5171x over previous
import functools

import jax
import jax.numpy as jnp
from jax import lax
from jax.experimental import pallas as pl
from jax.experimental.pallas import tpu as pltpu
from jax.experimental.pallas import tpu_sc as plsc

B, LD, LQ, V, D, NF = 4096, 200, 20, 1000000, 64, 4
W = NF + D
NC, NS = 2, 16
NW = NC * NS
CH = 512
KI = CH // 128
R1 = B * LD
R2 = B * LQ
Q1 = R1 // CH // NW
Q2 = R2 // CH // NW

_mesh = plsc.VectorSubcoreMesh(core_axis_name="c", subcore_axis_name="s")


@functools.partial(
    pl.kernel,
    mesh=_mesh,
    compiler_params=pltpu.CompilerParams(use_tc_tiling_on_sc=False,
                                         needs_layout_passes=False),
    out_type=[
        jax.ShapeDtypeStruct((R1, D), jnp.float32),
        jax.ShapeDtypeStruct((R2, D), jnp.float32),
    ],
    scratch_types=[
        pltpu.VMEM((2, CH), jnp.int32),
        pltpu.VMEM((2, CH, D), jnp.float32),
        pltpu.SemaphoreType.DMA,
        pltpu.SemaphoreType.DMA,
    ],
)
def _emb_gather(x1_hbm, x2_hbm, tbl_hbm, out1_hbm, out2_hbm,
                idx_v, row_v, sem0, sem1):
    wid = lax.axis_index("s") * NC + lax.axis_index("c")
    sems = (sem0, sem1)

    def fire(idx_hbm, r0, p):
        pltpu.sync_copy(idx_hbm.at[pl.ds(r0, CH)], idx_v.at[p])
        for j in range(KI):
            pltpu.async_copy(
                tbl_hbm.at[idx_v.at[p, pl.ds(j * 128, 128)]],
                row_v.at[p, pl.ds(j * 128, 128)],
                sems[p],
            )

    def finish(out_hbm, q, p):
        for j in range(KI):
            pltpu.make_async_copy(
                tbl_hbm.at[idx_v.at[p, pl.ds(j * 128, 128)]],
                row_v.at[p, pl.ds(j * 128, 128)],
                sems[p],
            ).wait()
        pltpu.sync_copy(row_v.at[p], out_hbm.at[pl.ds(q * CH, CH)])

    def sweep(idx_hbm, out_hbm, q0, nq):
        fire(idx_hbm, q0 * CH, 0)

        def pair(t, carry):
            q = q0 + 2 * t
            fire(idx_hbm, (q + 1) * CH, 1)
            finish(out_hbm, q, 0)

            @pl.when(q + 2 < q0 + nq)
            def _():
                fire(idx_hbm, (q + 2) * CH, 0)

            finish(out_hbm, q + 1, 1)
            return carry

        lax.fori_loop(0, nq // 2, pair, 0)

        @pl.when(nq % 2 == 1)
        def _():
            finish(out_hbm, q0 + nq - 1, 0)

    wid_q1 = wid * Q1
    wid_q2 = wid * Q2
    sweep(x1_hbm, out1_hbm, wid_q1, Q1)
    sweep(x2_hbm, out2_hbm, wid_q2, Q2)


def kernel(x1, x1_f, x1_pos, x1_ner, x1_mask, x2, x2_mask, sent_lens, emb_table):
    del x1_pos, x1_ner, x1_mask, x2_mask, sent_lens
    e1, e2 = _emb_gather(x1.reshape(R1), x2.reshape(R2), emb_table)
    x1_all = jnp.concatenate([x1_f, e1.reshape(B, LD, D)], axis=-1)
    return x1_all, e2.reshape(B, LQ, D)
